# baseline (speedup 1.0000x reference)
import functools

import jax
import jax.numpy as jnp
from jax import lax
from jax.experimental import pallas as pl
from jax.experimental.pallas import tpu as pltpu
from jax.experimental.pallas import tpu_sc as plsc

_BATCH = 16384
_HIDDEN = 64
_NUM_EMB = 1000001

_NC = 2
_NS = 16
_NW = _NC * _NS
_NVEC = _BATCH // 16
_MAIN_BLOCKS = 7808
_BPT = _MAIN_BLOCKS // _NW
_TAIL_START = _MAIN_BLOCKS * 128
_TAIL_ROWS = _NUM_EMB - _TAIL_START
_CAP = 8
_RING = 64


def _make_gather():
    mesh = plsc.VectorSubcoreMesh(core_axis_name="c", subcore_axis_name="s")

    @functools.partial(
        pl.kernel,
        out_type=jax.ShapeDtypeStruct((_BATCH, _HIDDEN), jnp.float32),
        mesh=mesh,
        scratch_types=[
            pltpu.VMEM((128, 16), jnp.int32),
            pltpu.VMEM((_BPT * _CAP,), jnp.int32),
            pltpu.VMEM((_BPT * _CAP,), jnp.int32),
            pltpu.VMEM((_HIDDEN, 128), jnp.float32),
            pltpu.VMEM((_RING, _HIDDEN), jnp.float32),
            pltpu.SMEM((256,), jnp.int32),
            pltpu.SemaphoreType.DMA,
        ],
        compiler_params=pltpu.CompilerParams(needs_layout_passes=False),
    )
    def gather_kernel(labels_hbm, table_t_hbm, tail_hbm, out_hbm,
                      lab_v, br_v, bp_v, blk_v, ring_v, cnt_s, sem):
        wid = lax.axis_index("s") * _NC + lax.axis_index("c")
        lo = wid * _BPT
        lane_iota = lax.iota(jnp.int32, 16)
        mask0 = lane_iota == 0

        def zero_counts(i, c):
            cnt_s[i] = 0
            return c
        lax.fori_loop(0, _BPT, zero_counts, 0)

        def extract_row(r, pos, slot):
            lane = jnp.broadcast_to(r & 127, (16,))
            for cg in range(_HIDDEN // 16):
                feats = plsc.load_gather(
                    blk_v, [cg * 16 + lane_iota, lane])
                plsc.store_scatter(
                    ring_v,
                    [jnp.broadcast_to(slot, (16,)), cg * 16 + lane_iota],
                    feats)
            pltpu.async_copy(
                ring_v.at[pl.ds(slot, 1)], out_hbm.at[pl.ds(pos, 1)], sem)

        def scan_chunk(ch, carry):
            pltpu.sync_copy(labels_hbm.at[pl.ds(ch * 128, 128)], lab_v)
            return lax.fori_loop(0, 128, functools.partial(scan_vec, ch),
                                 carry)

        def scan_vec(ch, vi, carry):
            h_tail = carry
            v = ch * 128 + vi
            vec = lab_v[vi]
            t0v = lax.shift_right_logical(vec, 7)
            m_main = (t0v >= lo) & (t0v < lo + _BPT)
            m_tail = (vec >= _TAIL_START) & (
                (jnp.broadcast_to(v * 16, (16,)) + lane_iota) % _NW == wid)
            n_any = plsc.all_reduce_population_count(m_main | m_tail)[0]

            @pl.when(n_any > 0)
            def _():
                for ln in range(16):
                    r = vec[ln]
                    t0 = r >> 7
                    pos = v * 16 + ln

                    @pl.when((t0 >= lo) & (t0 < lo + _BPT))
                    def _():
                        local = t0 - lo
                        c = cnt_s[local]
                        cnt_s[local] = c + 1

                        @pl.when(c < _CAP)
                        def _():
                            addr = jnp.broadcast_to(local * _CAP + c, (16,))
                            plsc.store_scatter(
                                br_v, [addr], jnp.broadcast_to(r, (16,)),
                                mask=mask0)
                            plsc.store_scatter(
                                bp_v, [addr], jnp.broadcast_to(pos, (16,)),
                                mask=mask0)

                        @pl.when(c >= _CAP)
                        def _():
                            pltpu.sync_copy(
                                table_t_hbm.at[:, pl.ds(t0 * 128, 128)],
                                blk_v)
                            extract_row(r, pos, jnp.int32(0))
                            pltpu.make_async_copy(
                                tail_hbm.at[pl.ds(0, 1)],
                                ring_v.at[pl.ds(0, 1)], sem).wait()

                    @pl.when((r >= _TAIL_START) & (pos % _NW == wid))
                    def _():
                        pltpu.sync_copy(
                            tail_hbm.at[pl.ds(r - _TAIL_START, 1)],
                            ring_v.at[pl.ds(0, 1)])
                        pltpu.async_copy(
                            ring_v.at[pl.ds(0, 1)],
                            out_hbm.at[pl.ds(pos, 1)], sem)
                        pltpu.make_async_copy(
                            tail_hbm.at[pl.ds(0, 1)],
                            ring_v.at[pl.ds(0, 1)], sem).wait()

            n_tail = plsc.all_reduce_population_count(m_tail)[0]
            return h_tail + n_tail

        n_tail_mine = lax.fori_loop(0, _NVEC // 128, scan_chunk,
                                    jnp.int32(0))

        def unit_wait(i, c):
            pltpu.make_async_copy(
                tail_hbm.at[pl.ds(0, 1)], ring_v.at[pl.ds(0, 1)], sem).wait()
            return c

        def do_block(k, carry):
            h_base, drained = carry
            cnt = jnp.minimum(cnt_s[k], _CAP)

            @pl.when(h_base - drained >= _RING - 2 * _CAP)
            def _():
                lax.fori_loop(0, h_base - drained, unit_wait, 0)

            drained = jnp.where(h_base - drained >= _RING - 2 * _CAP,
                                h_base, drained)

            @pl.when(cnt > 0)
            def _():
                pltpu.sync_copy(
                    table_t_hbm.at[:, pl.ds((lo + k) * 128, 128)], blk_v)
                for j in range(_CAP):
                    @pl.when(j < cnt)
                    def _():
                        h = h_base + j
                        addr = jnp.broadcast_to(k * _CAP + j, (16,))
                        r = plsc.load_gather(br_v, [addr])[0]
                        pos = plsc.load_gather(bp_v, [addr])[0]
                        extract_row(r, pos, h % _RING)

            return h_base + cnt, drained

        h_total, drained = lax.fori_loop(
            0, _BPT, do_block, (jnp.int32(0), jnp.int32(0)))

        del n_tail_mine
        lax.fori_loop(0, h_total - drained, unit_wait, 0)

    return gather_kernel


_gather = _make_gather()


def kernel(labels, embedding_table):
    labels2d = labels.astype(jnp.int32).reshape(_NVEC, 16)
    tail = lax.slice_in_dim(embedding_table, _TAIL_START, _NUM_EMB, axis=0)
    return _gather(labels2d, embedding_table.T, tail)


# speedup vs baseline: 1.2180x; 1.2180x over previous
import functools

import jax
import jax.numpy as jnp
from jax import lax
from jax.experimental import pallas as pl
from jax.experimental.pallas import tpu as pltpu
from jax.experimental.pallas import tpu_sc as plsc

_BATCH = 16384
_HIDDEN = 64
_NUM_EMB = 1000001

_NC = 2
_NS = 16
_NW = _NC * _NS
_NVEC = _BATCH // 16
_MAIN_BLOCKS = 7808
_BPT = _MAIN_BLOCKS // _NW
_TAIL_START = _MAIN_BLOCKS * 128
_TAIL_ROWS = _NUM_EMB - _TAIL_START
_CAP = 8
_RING = 64


def _make_gather():
    mesh = plsc.VectorSubcoreMesh(core_axis_name="c", subcore_axis_name="s")

    @functools.partial(
        pl.kernel,
        out_type=jax.ShapeDtypeStruct((_BATCH, _HIDDEN), jnp.float32),
        mesh=mesh,
        scratch_types=[
            pltpu.VMEM((128, 16), jnp.int32),
            pltpu.VMEM((_BPT * _CAP,), jnp.int32),
            pltpu.VMEM((_BPT * _CAP,), jnp.int32),
            pltpu.VMEM((_HIDDEN, 128), jnp.float32),
            pltpu.VMEM((_HIDDEN, 128), jnp.float32),
            pltpu.VMEM((_RING, _HIDDEN), jnp.float32),
            pltpu.SMEM((256,), jnp.int32),
            pltpu.SemaphoreType.DMA,
            pltpu.SemaphoreType.DMA,
            pltpu.SemaphoreType.DMA,
        ],
        compiler_params=pltpu.CompilerParams(needs_layout_passes=False),
    )
    def gather_kernel(labels_hbm, table_t_hbm, tail_hbm, out_hbm,
                      lab_v, br_v, bp_v, blk_a, blk_b, ring_v, cnt_s,
                      sem, sem_a, sem_b):
        wid = lax.axis_index("s") * _NC + lax.axis_index("c")
        lo = wid * _BPT
        lane_iota = lax.iota(jnp.int32, 16)
        mask0 = lane_iota == 0

        def zero_counts(i, c):
            cnt_s[i] = 0
            return c
        lax.fori_loop(0, _BPT, zero_counts, 0)

        def extract_row(blk_v, r, pos, slot):
            lane = jnp.broadcast_to(r & 127, (16,))
            for cg in range(_HIDDEN // 16):
                feats = plsc.load_gather(
                    blk_v, [cg * 16 + lane_iota, lane])
                plsc.store_scatter(
                    ring_v,
                    [jnp.broadcast_to(slot, (16,)), cg * 16 + lane_iota],
                    feats)
            pltpu.async_copy(
                ring_v.at[pl.ds(slot, 1)], out_hbm.at[pl.ds(pos, 1)], sem)

        def scan_chunk(ch, carry):
            pltpu.sync_copy(labels_hbm.at[pl.ds(ch * 128, 128)], lab_v)
            return lax.fori_loop(0, 128, functools.partial(scan_vec, ch),
                                 carry)

        def scan_vec(ch, vi, carry):
            h_tail = carry
            v = ch * 128 + vi
            vec = lab_v[vi]
            t0v = lax.shift_right_logical(vec, 7)
            m_main = (t0v >= lo) & (t0v < lo + _BPT)
            m_tail = (vec >= _TAIL_START) & (
                (jnp.broadcast_to(v * 16, (16,)) + lane_iota) % _NW == wid)
            n_any = plsc.all_reduce_population_count(m_main | m_tail)[0]

            @pl.when(n_any > 0)
            def _():
                for ln in range(16):
                    r = vec[ln]
                    t0 = r >> 7
                    pos = v * 16 + ln

                    @pl.when((t0 >= lo) & (t0 < lo + _BPT))
                    def _():
                        local = t0 - lo
                        c = cnt_s[local]
                        cnt_s[local] = c + 1

                        @pl.when(c < _CAP)
                        def _():
                            addr = jnp.broadcast_to(local * _CAP + c, (16,))
                            plsc.store_scatter(
                                br_v, [addr], jnp.broadcast_to(r, (16,)),
                                mask=mask0)
                            plsc.store_scatter(
                                bp_v, [addr], jnp.broadcast_to(pos, (16,)),
                                mask=mask0)

                        @pl.when(c >= _CAP)
                        def _():
                            pltpu.sync_copy(
                                table_t_hbm.at[:, pl.ds(t0 * 128, 128)],
                                blk_a)
                            extract_row(blk_a, r, pos, jnp.int32(0))
                            pltpu.make_async_copy(
                                tail_hbm.at[pl.ds(0, 1)],
                                ring_v.at[pl.ds(0, 1)], sem).wait()

                    @pl.when((r >= _TAIL_START) & (pos % _NW == wid))
                    def _():
                        pltpu.sync_copy(
                            tail_hbm.at[pl.ds(r - _TAIL_START, 1)],
                            ring_v.at[pl.ds(0, 1)])
                        pltpu.async_copy(
                            ring_v.at[pl.ds(0, 1)],
                            out_hbm.at[pl.ds(pos, 1)], sem)
                        pltpu.make_async_copy(
                            tail_hbm.at[pl.ds(0, 1)],
                            ring_v.at[pl.ds(0, 1)], sem).wait()

            n_tail = plsc.all_reduce_population_count(m_tail)[0]
            return h_tail + n_tail

        n_tail_mine = lax.fori_loop(0, _NVEC // 128, scan_chunk,
                                    jnp.int32(0))

        def unit_wait(i, c):
            pltpu.make_async_copy(
                tail_hbm.at[pl.ds(0, 1)], ring_v.at[pl.ds(0, 1)], sem).wait()
            return c

        def fetch(k, blk, fsem):
            pltpu.async_copy(
                table_t_hbm.at[:, pl.ds((lo + k) * 128, 128)], blk, fsem)

        def fetch_wait(blk, fsem):
            pltpu.make_async_copy(
                table_t_hbm.at[:, pl.ds(lo * 128, 128)], blk, fsem).wait()

        def handle_block(k, blk, h_base, drained):
            cnt = jnp.minimum(cnt_s[k], _CAP)

            @pl.when(h_base - drained >= _RING - 2 * _CAP)
            def _():
                lax.fori_loop(0, h_base - drained, unit_wait, 0)

            drained = jnp.where(h_base - drained >= _RING - 2 * _CAP,
                                h_base, drained)

            @pl.when(cnt > 0)
            def _():
                for j in range(_CAP):
                    @pl.when(j < cnt)
                    def _():
                        h = h_base + j
                        addr = jnp.broadcast_to(k * _CAP + j, (16,))
                        r = plsc.load_gather(br_v, [addr])[0]
                        pos = plsc.load_gather(bp_v, [addr])[0]
                        extract_row(blk, r, pos, h % _RING)

            return h_base + cnt, drained

        fetch(0, blk_a, sem_a)
        fetch(1, blk_b, sem_b)

        def do_pair(kk, carry):
            h_base, drained = carry
            for b, (blk, fsem) in enumerate(((blk_a, sem_a), (blk_b, sem_b))):
                k = kk * 2 + b
                fetch_wait(blk, fsem)
                h_base, drained = handle_block(k, blk, h_base, drained)

                @pl.when(k + 2 < _BPT)
                def _():
                    fetch(k + 2, blk, fsem)

            return h_base, drained

        h_total, drained = lax.fori_loop(
            0, _BPT // 2, do_pair, (jnp.int32(0), jnp.int32(0)))

        del n_tail_mine
        lax.fori_loop(0, h_total - drained, unit_wait, 0)

    return gather_kernel


_gather = _make_gather()


def kernel(labels, embedding_table):
    labels2d = labels.astype(jnp.int32).reshape(_NVEC, 16)
    tail = lax.slice_in_dim(embedding_table, _TAIL_START, _NUM_EMB, axis=0)
    return _gather(labels2d, embedding_table.T, tail)


# speedup vs baseline: 2.5461x; 2.0903x over previous
import functools

import jax
import jax.numpy as jnp
from jax import lax
from jax.experimental import pallas as pl
from jax.experimental.pallas import tpu as pltpu
from jax.experimental.pallas import tpu_sc as plsc

_BATCH = 16384
_HIDDEN = 64
_NUM_EMB = 1000001

_NC = 2
_NS = 16
_NW = _NC * _NS
_NVEC = _BATCH // 16
_MAIN_BLOCKS = 7808
_BPT = _MAIN_BLOCKS // _NW
_TAIL_START = _MAIN_BLOCKS * 128
_TAIL_ROWS = _NUM_EMB - _TAIL_START
_CAP = 8
_RING = 64


def _make_gather():
    mesh = plsc.VectorSubcoreMesh(core_axis_name="c", subcore_axis_name="s")

    @functools.partial(
        pl.kernel,
        out_type=jax.ShapeDtypeStruct((_BATCH, _HIDDEN), jnp.float32),
        mesh=mesh,
        scratch_types=[
            pltpu.VMEM((128, 16), jnp.int32),
            pltpu.VMEM((_BPT * _CAP,), jnp.int32),
            pltpu.VMEM((_BPT * _CAP,), jnp.int32),
            pltpu.VMEM((_HIDDEN, 128), jnp.float32),
            pltpu.VMEM((_HIDDEN, 128), jnp.float32),
            pltpu.VMEM((_RING, _HIDDEN), jnp.float32),
            pltpu.SMEM((256,), jnp.int32),
            pltpu.SemaphoreType.DMA,
            pltpu.SemaphoreType.DMA,
            pltpu.SemaphoreType.DMA,
        ],
        compiler_params=pltpu.CompilerParams(needs_layout_passes=False),
    )
    def gather_kernel(labels_hbm, table_t_hbm, tail_hbm, out_hbm,
                      lab_v, br_v, bp_v, blk_a, blk_b, ring_v, cnt_s,
                      sem, sem_a, sem_b):
        wid = lax.axis_index("s") * _NC + lax.axis_index("c")
        lo = wid * _BPT
        lane_iota = lax.iota(jnp.int32, 16)
        mask0 = lane_iota == 0

        def zero_counts(i, c):
            cnt_s[i] = 0
            return c
        lax.fori_loop(0, _BPT, zero_counts, 0)

        def extract_row(blk_v, r, pos, slot):
            lane = jnp.broadcast_to(r & 127, (16,))
            for cg in range(_HIDDEN // 16):
                feats = plsc.load_gather(
                    blk_v, [cg * 16 + lane_iota, lane])
                plsc.store_scatter(
                    ring_v,
                    [jnp.broadcast_to(slot, (16,)), cg * 16 + lane_iota],
                    feats)
            pltpu.async_copy(
                ring_v.at[pl.ds(slot, 1)], out_hbm.at[pl.ds(pos, 1)], sem)

        def scan_chunk(ch, carry):
            pltpu.sync_copy(labels_hbm.at[pl.ds(ch * 128, 128)], lab_v)
            return lax.fori_loop(0, 128, functools.partial(scan_vec, ch),
                                 carry)

        def scan_vec(ch, vi, carry):
            v = ch * 128 + vi
            vec = lab_v[vi]
            t0v = lax.shift_right_logical(vec, 7)
            m_main = (t0v >= lo) & (t0v < lo + _BPT)
            m_tail = (vec >= _TAIL_START) & (
                (jnp.broadcast_to(v * 16, (16,)) + lane_iota) % _NW == wid)

            def has_hits(m):
                return plsc.all_reduce_population_count(m)[0] > 0

            def handle_hit(m):
                ln = plsc.all_reduce_ffs(m)[0]
                sel = lane_iota == ln
                r = jnp.sum(jnp.where(sel, vec, 0))
                t0 = r >> 7
                pos = v * 16 + ln

                @pl.when(r < _TAIL_START)
                def _():
                    local = t0 - lo
                    c = cnt_s[local]
                    cnt_s[local] = c + 1

                    @pl.when(c < _CAP)
                    def _():
                        addr = jnp.broadcast_to(local * _CAP + c, (16,))
                        plsc.store_scatter(
                            br_v, [addr], jnp.broadcast_to(r, (16,)),
                            mask=mask0)
                        plsc.store_scatter(
                            bp_v, [addr], jnp.broadcast_to(pos, (16,)),
                            mask=mask0)

                    @pl.when(c >= _CAP)
                    def _():
                        pltpu.sync_copy(
                            table_t_hbm.at[:, pl.ds(t0 * 128, 128)],
                            blk_a)
                        extract_row(blk_a, r, pos, jnp.int32(0))
                        pltpu.make_async_copy(
                            tail_hbm.at[pl.ds(0, 1)],
                            ring_v.at[pl.ds(0, 1)], sem).wait()

                @pl.when(r >= _TAIL_START)
                def _():
                    pltpu.sync_copy(
                        tail_hbm.at[pl.ds(r - _TAIL_START, 1)],
                        ring_v.at[pl.ds(0, 1)])
                    pltpu.async_copy(
                        ring_v.at[pl.ds(0, 1)],
                        out_hbm.at[pl.ds(pos, 1)], sem)
                    pltpu.make_async_copy(
                        tail_hbm.at[pl.ds(0, 1)],
                        ring_v.at[pl.ds(0, 1)], sem).wait()

                return m & jnp.logical_not(sel)

            lax.while_loop(has_hits, handle_hit, m_main | m_tail)
            return carry

        lax.fori_loop(0, _NVEC // 128, scan_chunk, jnp.int32(0))
        n_tail_mine = None

        def unit_wait(i, c):
            pltpu.make_async_copy(
                tail_hbm.at[pl.ds(0, 1)], ring_v.at[pl.ds(0, 1)], sem).wait()
            return c

        def fetch(k, blk, fsem):
            pltpu.async_copy(
                table_t_hbm.at[:, pl.ds((lo + k) * 128, 128)], blk, fsem)

        def fetch_wait(blk, fsem):
            pltpu.make_async_copy(
                table_t_hbm.at[:, pl.ds(lo * 128, 128)], blk, fsem).wait()

        def handle_block(k, blk, h_base, drained):
            cnt = jnp.minimum(cnt_s[k], _CAP)

            @pl.when(h_base - drained >= _RING - 2 * _CAP)
            def _():
                lax.fori_loop(0, h_base - drained, unit_wait, 0)

            drained = jnp.where(h_base - drained >= _RING - 2 * _CAP,
                                h_base, drained)

            def do_hit(j, c):
                h = h_base + j
                addr = jnp.broadcast_to(k * _CAP + j, (16,))
                r = plsc.load_gather(br_v, [addr])[0]
                pos = plsc.load_gather(bp_v, [addr])[0]
                extract_row(blk, r, pos, h % _RING)
                return c
            lax.fori_loop(0, cnt, do_hit, 0)

            return h_base + cnt, drained

        fetch(0, blk_a, sem_a)
        fetch(1, blk_b, sem_b)

        def do_pair(kk, carry):
            h_base, drained = carry
            for b, (blk, fsem) in enumerate(((blk_a, sem_a), (blk_b, sem_b))):
                k = kk * 2 + b
                fetch_wait(blk, fsem)
                h_base, drained = handle_block(k, blk, h_base, drained)

                @pl.when(k + 2 < _BPT)
                def _():
                    fetch(k + 2, blk, fsem)

            return h_base, drained

        h_total, drained = lax.fori_loop(
            0, _BPT // 2, do_pair, (jnp.int32(0), jnp.int32(0)))

        del n_tail_mine
        lax.fori_loop(0, h_total - drained, unit_wait, 0)

    return gather_kernel


_gather = _make_gather()


def kernel(labels, embedding_table):
    labels2d = labels.astype(jnp.int32).reshape(_NVEC, 16)
    tail = lax.slice_in_dim(embedding_table, _TAIL_START, _NUM_EMB, axis=0)
    return _gather(labels2d, embedding_table.T, tail)


# speedup vs baseline: 2.6192x; 1.0287x over previous
import functools

import jax
import jax.numpy as jnp
from jax import lax
from jax.experimental import pallas as pl
from jax.experimental.pallas import tpu as pltpu
from jax.experimental.pallas import tpu_sc as plsc

_BATCH = 16384
_HIDDEN = 64
_NUM_EMB = 1000001

_NC = 2
_NS = 16
_NW = _NC * _NS
_NVEC = _BATCH // 16
_MAIN_BLOCKS = 7808
_BPT = _MAIN_BLOCKS // _NW
_TAIL_START = _MAIN_BLOCKS * 128
_TAIL_ROWS = _NUM_EMB - _TAIL_START
_CAP = 8
_RING = 64


def _make_gather():
    mesh = plsc.VectorSubcoreMesh(core_axis_name="c", subcore_axis_name="s")

    @functools.partial(
        pl.kernel,
        out_type=jax.ShapeDtypeStruct((_BATCH, _HIDDEN), jnp.float32),
        mesh=mesh,
        scratch_types=[
            pltpu.VMEM((128, 16), jnp.int32),
            pltpu.VMEM((_BPT * _CAP,), jnp.int32),
            pltpu.VMEM((_BPT * _CAP,), jnp.int32),
            pltpu.VMEM((_HIDDEN, 128), jnp.float32),
            pltpu.VMEM((_HIDDEN, 128), jnp.float32),
            pltpu.VMEM((_RING, _HIDDEN), jnp.float32),
            pltpu.SMEM((256,), jnp.int32),
            pltpu.SemaphoreType.DMA,
            pltpu.SemaphoreType.DMA,
            pltpu.SemaphoreType.DMA,
        ],
        compiler_params=pltpu.CompilerParams(needs_layout_passes=False),
    )
    def gather_kernel(labels_hbm, table_t_hbm, tail_hbm, out_hbm,
                      lab_v, br_v, bp_v, blk_a, blk_b, ring_v, cnt_s,
                      sem, sem_a, sem_b):
        wid = lax.axis_index("s") * _NC + lax.axis_index("c")
        lo = wid * _BPT
        lane_iota = lax.iota(jnp.int32, 16)
        mask0 = lane_iota == 0

        def zero_counts(i, c):
            cnt_s[i] = 0
            return c
        lax.fori_loop(0, _BPT, zero_counts, 0)

        def extract_row(blk_v, r, pos, slot):
            lane = jnp.broadcast_to(r & 127, (16,))
            for cg in range(_HIDDEN // 16):
                feats = plsc.load_gather(
                    blk_v, [cg * 16 + lane_iota, lane])
                plsc.store_scatter(
                    ring_v,
                    [jnp.broadcast_to(slot, (16,)), cg * 16 + lane_iota],
                    feats)
            pltpu.async_copy(
                ring_v.at[pl.ds(slot, 1)], out_hbm.at[pl.ds(pos, 1)], sem)

        def scan_chunk(ch, carry):
            pltpu.sync_copy(labels_hbm.at[pl.ds(ch * 128, 128)], lab_v)
            return lax.fori_loop(0, 128, functools.partial(scan_vec, ch),
                                 carry)

        def scan_vec(ch, vi, carry):
            v = ch * 128 + vi
            vec = lab_v[vi]
            t0v = lax.shift_right_logical(vec, 7)
            m_main = (t0v >= lo) & (t0v < lo + _BPT)
            m_tail = (vec >= _TAIL_START) & (
                ((jnp.broadcast_to(v * 16, (16,)) + lane_iota) & (_NW - 1))
                == wid)

            def handle_hit(i, m):
                ln = plsc.all_reduce_ffs(m)[0]
                sel = lane_iota == ln
                r = jnp.sum(jnp.where(sel, vec, 0))
                t0 = r >> 7
                pos = v * 16 + ln

                @pl.when(r < _TAIL_START)
                def _():
                    local = t0 - lo
                    c = cnt_s[local]
                    cnt_s[local] = c + 1

                    @pl.when(c < _CAP)
                    def _():
                        addr = jnp.broadcast_to(local * _CAP + c, (16,))
                        plsc.store_scatter(
                            br_v, [addr], jnp.broadcast_to(r, (16,)),
                            mask=mask0)
                        plsc.store_scatter(
                            bp_v, [addr], jnp.broadcast_to(pos, (16,)),
                            mask=mask0)

                    @pl.when(c >= _CAP)
                    def _():
                        pltpu.sync_copy(
                            table_t_hbm.at[:, pl.ds(t0 * 128, 128)],
                            blk_a)
                        extract_row(blk_a, r, pos, jnp.int32(0))
                        pltpu.make_async_copy(
                            tail_hbm.at[pl.ds(0, 1)],
                            ring_v.at[pl.ds(0, 1)], sem).wait()

                @pl.when(r >= _TAIL_START)
                def _():
                    pltpu.sync_copy(
                        tail_hbm.at[pl.ds(r - _TAIL_START, 1)],
                        ring_v.at[pl.ds(0, 1)])
                    pltpu.async_copy(
                        ring_v.at[pl.ds(0, 1)],
                        out_hbm.at[pl.ds(pos, 1)], sem)
                    pltpu.make_async_copy(
                        tail_hbm.at[pl.ds(0, 1)],
                        ring_v.at[pl.ds(0, 1)], sem).wait()

                return m & jnp.logical_not(sel)

            m = m_main | m_tail
            n = plsc.all_reduce_population_count(m)[0]
            lax.fori_loop(0, n, handle_hit, m)
            return carry

        lax.fori_loop(0, _NVEC // 128, scan_chunk, jnp.int32(0))
        n_tail_mine = None

        def unit_wait(i, c):
            pltpu.make_async_copy(
                tail_hbm.at[pl.ds(0, 1)], ring_v.at[pl.ds(0, 1)], sem).wait()
            return c

        def fetch(k, blk, fsem):
            pltpu.async_copy(
                table_t_hbm.at[:, pl.ds((lo + k) * 128, 128)], blk, fsem)

        def fetch_wait(blk, fsem):
            pltpu.make_async_copy(
                table_t_hbm.at[:, pl.ds(lo * 128, 128)], blk, fsem).wait()

        def handle_block(k, blk, h_base, drained):
            cnt = jnp.minimum(cnt_s[k], _CAP)

            @pl.when(h_base - drained >= _RING - 2 * _CAP)
            def _():
                lax.fori_loop(0, h_base - drained, unit_wait, 0)

            drained = jnp.where(h_base - drained >= _RING - 2 * _CAP,
                                h_base, drained)

            def do_hit(j, c):
                h = h_base + j
                addr = jnp.broadcast_to(k * _CAP + j, (16,))
                r = plsc.load_gather(br_v, [addr])[0]
                pos = plsc.load_gather(bp_v, [addr])[0]
                extract_row(blk, r, pos, h % _RING)
                return c
            lax.fori_loop(0, cnt, do_hit, 0)

            return h_base + cnt, drained

        fetch(0, blk_a, sem_a)
        fetch(1, blk_b, sem_b)

        def do_pair(kk, carry):
            h_base, drained = carry
            for b, (blk, fsem) in enumerate(((blk_a, sem_a), (blk_b, sem_b))):
                k = kk * 2 + b
                fetch_wait(blk, fsem)
                h_base, drained = handle_block(k, blk, h_base, drained)

                @pl.when(k + 2 < _BPT)
                def _():
                    fetch(k + 2, blk, fsem)

            return h_base, drained

        h_total, drained = lax.fori_loop(
            0, _BPT // 2, do_pair, (jnp.int32(0), jnp.int32(0)))

        del n_tail_mine
        lax.fori_loop(0, h_total - drained, unit_wait, 0)

    return gather_kernel


_gather = _make_gather()


def kernel(labels, embedding_table):
    labels2d = labels.astype(jnp.int32).reshape(_NVEC, 16)
    tail = lax.slice_in_dim(embedding_table, _TAIL_START, _NUM_EMB, axis=0)
    return _gather(labels2d, embedding_table.T, tail)


# speedup vs baseline: 2.6207x; 1.0006x over previous
import functools

import jax
import jax.numpy as jnp
from jax import lax
from jax.experimental import pallas as pl
from jax.experimental.pallas import tpu as pltpu
from jax.experimental.pallas import tpu_sc as plsc

_BATCH = 16384
_HIDDEN = 64
_NUM_EMB = 1000001

_NC = 2
_NS = 16
_NW = _NC * _NS
_NVEC = _BATCH // 16
_MAIN_BLOCKS = 7808
_BPT = _MAIN_BLOCKS // _NW
_TAIL_START = _MAIN_BLOCKS * 128
_TAIL_ROWS = _NUM_EMB - _TAIL_START
_CAP = 8
_RING = 64


def _make_gather():
    mesh = plsc.VectorSubcoreMesh(core_axis_name="c", subcore_axis_name="s")

    @functools.partial(
        pl.kernel,
        out_type=jax.ShapeDtypeStruct((_BATCH, _HIDDEN), jnp.float32),
        mesh=mesh,
        scratch_types=[
            pltpu.VMEM((128, 16), jnp.int32),
            pltpu.VMEM((_BPT * _CAP,), jnp.int32),
            pltpu.VMEM((_BPT * _CAP,), jnp.int32),
            pltpu.VMEM((_HIDDEN, 128), jnp.float32),
            pltpu.VMEM((_HIDDEN, 128), jnp.float32),
            pltpu.VMEM((_RING, _HIDDEN), jnp.float32),
            pltpu.SMEM((256,), jnp.int32),
            pltpu.SemaphoreType.DMA,
            pltpu.SemaphoreType.DMA,
            pltpu.SemaphoreType.DMA,
        ],
        compiler_params=pltpu.CompilerParams(needs_layout_passes=False),
    )
    def gather_kernel(labels_hbm, table_t_hbm, tail_hbm, out_hbm,
                      lab_v, br_v, bp_v, blk_a, blk_b, ring_v, cnt_s,
                      sem, sem_a, sem_b):
        wid = lax.axis_index("s") * _NC + lax.axis_index("c")
        lo = wid * _BPT
        lane_iota = lax.iota(jnp.int32, 16)
        mask0 = lane_iota == 0

        def zero_counts(i, c):
            cnt_s[i] = 0
            return c
        lax.fori_loop(0, _BPT, zero_counts, 0)

        def extract_row(blk_v, r, pos, slot):
            lane = jnp.broadcast_to(r & 127, (16,))
            for cg in range(_HIDDEN // 16):
                feats = plsc.load_gather(
                    blk_v, [cg * 16 + lane_iota, lane])
                plsc.store_scatter(
                    ring_v,
                    [jnp.broadcast_to(slot, (16,)), cg * 16 + lane_iota],
                    feats)
            pltpu.async_copy(
                ring_v.at[pl.ds(slot, 1)], out_hbm.at[pl.ds(pos, 1)], sem)

        def scan_chunk(ch, carry):
            pltpu.sync_copy(labels_hbm.at[pl.ds(ch * 128, 128)], lab_v)
            return lax.fori_loop(0, 128, functools.partial(scan_vec, ch),
                                 carry)

        def scan_vec(ch, vi, carry):
            v = ch * 128 + vi
            vec = lab_v[vi]
            t0v = lax.shift_right_logical(vec, 7)
            m_main = (t0v >= lo) & (t0v < lo + _BPT)
            m_tail = (vec >= _TAIL_START) & (
                ((jnp.broadcast_to(v * 16, (16,)) + lane_iota) & (_NW - 1))
                == wid)

            def handle_hit(i, m):
                ln = plsc.all_reduce_ffs(m)[0]
                sel = lane_iota == ln
                r = jnp.sum(jnp.where(sel, vec, 0))
                t0 = r >> 7
                pos = v * 16 + ln

                @pl.when(r < _TAIL_START)
                def _():
                    local = t0 - lo
                    c = cnt_s[local]
                    cnt_s[local] = c + 1

                    @pl.when(c < _CAP)
                    def _():
                        addr = jnp.broadcast_to(local * _CAP + c, (16,))
                        plsc.store_scatter(
                            br_v, [addr], jnp.broadcast_to(r, (16,)),
                            mask=mask0)
                        plsc.store_scatter(
                            bp_v, [addr], jnp.broadcast_to(pos, (16,)),
                            mask=mask0)

                    @pl.when(c >= _CAP)
                    def _():
                        pltpu.sync_copy(
                            table_t_hbm.at[:, pl.ds(t0 * 128, 128)],
                            blk_a)
                        extract_row(blk_a, r, pos, jnp.int32(0))
                        pltpu.make_async_copy(
                            tail_hbm.at[pl.ds(0, 1)],
                            ring_v.at[pl.ds(0, 1)], sem).wait()

                @pl.when(r >= _TAIL_START)
                def _():
                    pltpu.sync_copy(
                        tail_hbm.at[pl.ds(r - _TAIL_START, 1)],
                        ring_v.at[pl.ds(0, 1)])
                    pltpu.async_copy(
                        ring_v.at[pl.ds(0, 1)],
                        out_hbm.at[pl.ds(pos, 1)], sem)
                    pltpu.make_async_copy(
                        tail_hbm.at[pl.ds(0, 1)],
                        ring_v.at[pl.ds(0, 1)], sem).wait()

                return m & jnp.logical_not(sel)

            m = m_main | m_tail
            n = plsc.all_reduce_population_count(m)[0]
            lax.fori_loop(0, n, handle_hit, m)
            return carry

        lax.fori_loop(0, _NVEC // 128, scan_chunk, jnp.int32(0))

        def unit_wait(i, c):
            pltpu.make_async_copy(
                tail_hbm.at[pl.ds(0, 1)], ring_v.at[pl.ds(0, 1)], sem).wait()
            return c

        def fetch(k, blk, fsem):
            pltpu.async_copy(
                table_t_hbm.at[:, pl.ds((lo + k) * 128, 128)], blk, fsem)

        def fetch_wait(blk, fsem):
            pltpu.make_async_copy(
                table_t_hbm.at[:, pl.ds(lo * 128, 128)], blk, fsem).wait()

        def handle_block(k, blk, h_base, drained):
            cnt = jnp.minimum(cnt_s[k], _CAP)

            @pl.when(h_base - drained >= _RING - 2 * _CAP)
            def _():
                lax.fori_loop(0, h_base - drained, unit_wait, 0)

            drained = jnp.where(h_base - drained >= _RING - 2 * _CAP,
                                h_base, drained)

            def do_hit(j, c):
                h = h_base + j
                addr = jnp.broadcast_to(k * _CAP + j, (16,))
                r = plsc.load_gather(br_v, [addr])[0]
                pos = plsc.load_gather(bp_v, [addr])[0]
                extract_row(blk, r, pos, h % _RING)
                return c
            lax.fori_loop(0, cnt, do_hit, 0)

            return h_base + cnt, drained

        fetch(0, blk_a, sem_a)
        fetch(1, blk_b, sem_b)

        def do_pair(kk, carry):
            h_base, drained = carry
            for b, (blk, fsem) in enumerate(((blk_a, sem_a), (blk_b, sem_b))):
                k = kk * 2 + b
                fetch_wait(blk, fsem)
                h_base, drained = handle_block(k, blk, h_base, drained)

                @pl.when(k + 2 < _BPT)
                def _():
                    fetch(k + 2, blk, fsem)

            return h_base, drained

        h_total, drained = lax.fori_loop(
            0, _BPT // 2, do_pair, (jnp.int32(0), jnp.int32(0)))

        lax.fori_loop(0, h_total - drained, unit_wait, 0)

    return gather_kernel


_gather = _make_gather()


def kernel(labels, embedding_table):
    labels2d = labels.astype(jnp.int32).reshape(_NVEC, 16)
    tail = lax.slice_in_dim(embedding_table, _TAIL_START, _NUM_EMB, axis=0)
    return _gather(labels2d, embedding_table.T, tail)
